# baseline (speedup 1.0000x reference)
import functools

import jax
import jax.numpy as jnp
from jax import lax
from jax.experimental import pallas as pl
from jax.experimental.pallas import tpu as pltpu
from jax.experimental.pallas import tpu_sc as plsc

B, S, H = 2, 2048, 2048
E, TOP_K, R = 8, 2, 64
R2 = TOP_K * R

S_TILE = 512


def _expert_body(idx_ref, gate_ref, x_ref, wd0_ref, wd1_ref, wu0_ref, wu1_ref,
                 out_ref, wdc, wuc):
    b = pl.program_id(0)
    s = pl.program_id(1)

    @pl.when(s == 0)
    def _():
        wdc[:R, :] = wd0_ref[0].astype(jnp.bfloat16)
        wdc[R:, :] = wd1_ref[0].astype(jnp.bfloat16)
        wuc[:, :R] = wu0_ref[0].astype(jnp.bfloat16)
        wuc[:, R:] = wu1_ref[0].astype(jnp.bfloat16)

    g0 = gate_ref[b, 0]
    g1 = gate_ref[b, 1]
    col = lax.broadcasted_iota(jnp.int32, (1, R2), 1)
    gv = jnp.where(col < R, g0, g1)

    nt = (((1,), (1,)), ((), ()))
    xb = x_ref[0]
    h = lax.dot_general(xb.astype(jnp.bfloat16), wdc[...], nt,
                        preferred_element_type=jnp.float32)
    hg = (h * gv).astype(jnp.bfloat16)
    eo = lax.dot_general(hg, wuc[...], nt,
                         preferred_element_type=jnp.float32)
    out_ref[0] = xb + eo


@jax.jit
def _expert_call(x, wd_t, wu_t, idx_flat, gate_flat):
    grid = (B, S // S_TILE)
    spec = pltpu.PrefetchScalarGridSpec(
        num_scalar_prefetch=2,
        grid=grid,
        in_specs=[
            pl.BlockSpec((1, S_TILE, H), lambda b, s, idx, gate: (b, s, 0)),
            pl.BlockSpec((1, R, H), lambda b, s, idx, gate: (idx[b, 0], 0, 0)),
            pl.BlockSpec((1, R, H), lambda b, s, idx, gate: (idx[b, 1], 0, 0)),
            pl.BlockSpec((1, H, R), lambda b, s, idx, gate: (idx[b, 0], 0, 0)),
            pl.BlockSpec((1, H, R), lambda b, s, idx, gate: (idx[b, 1], 0, 0)),
        ],
        out_specs=pl.BlockSpec((1, S_TILE, H), lambda b, s, idx, gate: (b, s, 0)),
        scratch_shapes=[
            pltpu.VMEM((R2, H), jnp.bfloat16),
            pltpu.VMEM((H, R2), jnp.bfloat16),
        ],
    )
    return pl.pallas_call(
        _expert_body,
        grid_spec=spec,
        out_shape=jax.ShapeDtypeStruct((B, S, H), jnp.float32),
        compiler_params=pltpu.CompilerParams(
            dimension_semantics=("arbitrary", "arbitrary"),
        ),
    )(idx_flat, gate_flat, x, wd_t, wd_t, wu_t, wu_t)


def _route_jnp(x, router_w):
    cls = x[:, 0, :]
    logits = cls @ router_w.T
    topv, topi = lax.top_k(logits, TOP_K)
    gate = jax.nn.softmax(topv, axis=-1)
    return topi.reshape(-1).astype(jnp.int32), gate.reshape(-1)


def _lane_rotate(v, lane, sh):
    perm = lax.rem(lane + sh, 16)
    dnums = lax.GatherDimensionNumbers(
        offset_dims=(), collapsed_slice_dims=(0,), start_index_map=(0,))
    return lax.gather(v, perm[:, None], dnums, slice_sizes=(1,),
                      mode=lax.GatherScatterMode.PROMISE_IN_BOUNDS)


def _allreduce16(v, lane, op):
    for sh in (8, 4, 2, 1):
        v = op(v, _lane_rotate(v, lane, sh))
    return v


_UNROLL = 8


def _round_bf16(v):
    u = plsc.bitcast(v, jnp.int32)
    lsb = jnp.bitwise_and(lax.shift_right_logical(u, 16), 1)
    r = jnp.bitwise_and(u + 0x7FFF + lsb, jnp.int32(-65536))
    return plsc.bitcast(r, jnp.float32)


def _route_sc_body(x_hbm, rw_hbm, idx_out, gate_out,
                   cls_v, rw_row_v, logit_v, gath_v, idx_v, gate_v, shared):
    b = lax.axis_index("c")
    e = lax.axis_index("s")
    lane = lax.iota(jnp.int32, 16)

    @pl.when(e < E)
    def _():
        pltpu.sync_copy(x_hbm.at[b, 0], cls_v)
        pltpu.sync_copy(rw_hbm.at[e], rw_row_v)

        def body(j, accs):
            base = j * (16 * _UNROLL)
            accs = list(accs)
            for q in range(_UNROLL):
                off = base + q * 16
                accs[q % 4] = accs[q % 4] + (_round_bf16(cls_v[pl.ds(off, 16)]) *
                                             _round_bf16(rw_row_v[pl.ds(off, 16)]))
            return tuple(accs)

        z = jnp.zeros((16,), jnp.float32)
        a0, a1, a2, a3 = lax.fori_loop(0, H // (16 * _UNROLL), body,
                                       (z, z, z, z))
        logit_v[...] = _allreduce16((a0 + a1) + (a2 + a3), lane, jnp.add)
        pltpu.sync_copy(logit_v, shared.at[b, e])

    plsc.subcore_barrier()

    @pl.when(e == 0)
    def _():
        pltpu.sync_copy(shared.at[b], gath_v)
        logits = jnp.full((16,), -jnp.inf, dtype=jnp.float32)
        for e2 in range(E):
            logits = jnp.where(lane == e2, gath_v[e2], logits)
        m1 = _allreduce16(logits, lane, jnp.maximum)
        i1 = _allreduce16(jnp.where(logits == m1, lane, 16), lane, jnp.minimum)
        masked = jnp.where(lane == i1, -jnp.inf, logits)
        m2 = _allreduce16(masked, lane, jnp.maximum)
        i2 = _allreduce16(jnp.where(masked == m2, lane, 16), lane, jnp.minimum)
        e2v = jnp.exp(m2 - m1)
        denom = 1.0 + e2v
        zero_i = jnp.zeros((16,), jnp.int32)
        zero_f = jnp.zeros((16,), jnp.float32)
        idx_v[...] = jnp.where(lane == 0, i1, jnp.where(lane == 1, i2, zero_i))
        gate_v[...] = jnp.where(lane == 0, 1.0 / denom,
                                jnp.where(lane == 1, e2v / denom, zero_f))
        pltpu.sync_copy(idx_v, idx_out.at[b])
        pltpu.sync_copy(gate_v, gate_out.at[b])


@jax.jit
def _route_sc(x, router_w):
    return pl.kernel(
        _route_sc_body,
        out_type=(jax.ShapeDtypeStruct((B, 16), jnp.int32),
                  jax.ShapeDtypeStruct((B, 16), jnp.float32)),
        mesh=plsc.VectorSubcoreMesh(core_axis_name="c", subcore_axis_name="s"),
        compiler_params=pltpu.CompilerParams(needs_layout_passes=False),
        scratch_types=[
            pltpu.VMEM((H,), jnp.float32),
            pltpu.VMEM((H,), jnp.float32),
            pltpu.VMEM((16,), jnp.float32),
            pltpu.VMEM((E, 16), jnp.float32),
            pltpu.VMEM((16,), jnp.int32),
            pltpu.VMEM((16,), jnp.float32),
            pltpu.VMEM_SHARED((B, E, 16), jnp.float32),
        ],
    )(x, router_w)


def kernel(x, router_w, Wd, Wu):
    idx_rows, gate_rows = _route_sc(x, router_w)
    return _expert_call(x, Wd, Wu, idx_rows, gate_rows)


# speedup vs baseline: 1.3429x; 1.3429x over previous
import jax
import jax.numpy as jnp
from jax import lax
from jax.experimental import pallas as pl
from jax.experimental.pallas import tpu as pltpu

B, S, H = 2, 2048, 2048
E, TOP_K, R = 8, 2, 64
R2 = TOP_K * R

S_TILE = 512


def _body(x_ref, rw_ref, wd_any, wu_any, out_ref,
          wdc, wuc, gvs, wd0f, wd1f, wu0f, wu1f, sem):
    s = pl.program_id(1)
    col = lax.broadcasted_iota(jnp.int32, (1, R2), 1)

    @pl.when(s == 0)
    def _():
        cls = x_ref[0, 0:1, :].astype(jnp.bfloat16).astype(jnp.float32)
        rw = rw_ref[...].astype(jnp.bfloat16).astype(jnp.float32)
        lv = jnp.sum(cls * rw, axis=1, keepdims=True)
        erow = lax.broadcasted_iota(jnp.int32, (E, 1), 0)
        m1 = jnp.max(lv)
        i1 = jnp.min(jnp.where(lv == m1, erow, E))
        masked = jnp.where(erow == i1, -jnp.inf, lv)
        m2 = jnp.max(masked)
        i2 = jnp.min(jnp.where(masked == m2, erow, E))
        ev = jnp.exp(jnp.full((1, R2), m2 - m1, jnp.float32))
        gvs[...] = jnp.where(col < R, 1.0, ev) / (1.0 + ev)

        c0 = pltpu.make_async_copy(wd_any.at[i1], wd0f, sem)
        c0.start()
        c1 = pltpu.make_async_copy(wd_any.at[i2], wd1f, sem)
        c1.start()
        c2 = pltpu.make_async_copy(wu_any.at[i1], wu0f, sem)
        c2.start()
        c3 = pltpu.make_async_copy(wu_any.at[i2], wu1f, sem)
        c3.start()
        c0.wait()
        c1.wait()
        c2.wait()
        c3.wait()
        wdc[:R, :] = wd0f[...].astype(jnp.bfloat16)
        wdc[R:, :] = wd1f[...].astype(jnp.bfloat16)
        wuc[:, :R] = wu0f[...].astype(jnp.bfloat16)
        wuc[:, R:] = wu1f[...].astype(jnp.bfloat16)

    nt = (((1,), (1,)), ((), ()))
    xb = x_ref[0]
    h = lax.dot_general(xb.astype(jnp.bfloat16), wdc[...], nt,
                        preferred_element_type=jnp.float32)
    hg = (h * gvs[...]).astype(jnp.bfloat16)
    eo = lax.dot_general(hg, wuc[...], nt,
                         preferred_element_type=jnp.float32)
    out_ref[0] = xb + eo


@jax.jit
def _moe_call(x, router_w, Wd, Wu):
    grid = (B, S // S_TILE)
    return pl.pallas_call(
        _body,
        grid=grid,
        in_specs=[
            pl.BlockSpec((1, S_TILE, H), lambda b, s: (b, s, 0)),
            pl.BlockSpec((E, H), lambda b, s: (0, 0)),
            pl.BlockSpec(memory_space=pl.ANY),
            pl.BlockSpec(memory_space=pl.ANY),
        ],
        out_specs=pl.BlockSpec((1, S_TILE, H), lambda b, s: (b, s, 0)),
        scratch_shapes=[
            pltpu.VMEM((R2, H), jnp.bfloat16),
            pltpu.VMEM((H, R2), jnp.bfloat16),
            pltpu.VMEM((1, R2), jnp.float32),
            pltpu.VMEM((R, H), jnp.float32),
            pltpu.VMEM((R, H), jnp.float32),
            pltpu.VMEM((H, R), jnp.float32),
            pltpu.VMEM((H, R), jnp.float32),
            pltpu.SemaphoreType.DMA,
        ],
        out_shape=jax.ShapeDtypeStruct((B, S, H), jnp.float32),
        compiler_params=pltpu.CompilerParams(
            dimension_semantics=("arbitrary", "arbitrary"),
        ),
    )(x, router_w, Wd, Wu)


def kernel(x, router_w, Wd, Wu):
    return _moe_call(x, router_w, Wd, Wu)
